# Initial kernel scaffold; baseline (speedup 1.0000x reference)
import jax
import jax.numpy as jnp
from jax.experimental import pallas as pl


def kernel(x, edge_index, edge_attr, emb, W1, a1s, a1d, b1, W2, a2s, a2d, b2, W3, a3s, a3d, b3):
    raise NotImplementedError("write your pallas kernel here")


# speedup vs baseline: 8.9193x; 8.9193x over previous
import functools

import jax
import jax.numpy as jnp
from jax import lax
from jax.experimental import pallas as pl
from jax.experimental.pallas import tpu as pltpu
from jax.experimental.pallas import tpu_sc as plsc

N = 10000
E = 160000
D = 256
H = 128
NEG = 0.2
NS = 16
T = E // NS
B = 80
NBATCH = T // B
RPT = N // NS
BLK = 1000
EPS = 1e-16


def _proj_tail(hp, h_ref, s_ref, d_ref, as_ref, ad_ref):
    h_ref[0] = hp[:, :H]
    h_ref[1] = hp[:, H:]
    s_ref[...] = jnp.dot(hp, as_ref[...], preferred_element_type=jnp.float32)
    d_ref[...] = jnp.dot(hp, ad_ref[...], preferred_element_type=jnp.float32)


def _l1_body(x_ref, emb_ref, W_ref, as_ref, ad_ref, g_ref, h_ref, s_ref, d_ref):
    xb = x_ref[:, 0]
    oh = (xb[:, None] == lax.broadcasted_iota(jnp.int32, (BLK, 32), 1)).astype(jnp.float32)
    g = jnp.dot(oh, emb_ref[...], preferred_element_type=jnp.float32)
    g_ref[0] = g[:, :H]
    g_ref[1] = g[:, H:]
    hp = jnp.dot(g, W_ref[...], preferred_element_type=jnp.float32)
    _proj_tail(hp, h_ref, s_ref, d_ref, as_ref, ad_ref)


def _mid_body(num_ref, den_ref, b_ref, gp_ref, W_ref, as_ref, ad_ref,
              g_ref, h_ref, s_ref, d_ref):
    inv = 1.0 / (den_ref[...] + EPS)
    gl = jnp.maximum(num_ref[0] * inv + b_ref[0:1, :H], 0.0) + gp_ref[0]
    gr = jnp.maximum(num_ref[1] * inv + b_ref[0:1, H:], 0.0) + gp_ref[1]
    g_ref[0] = gl
    g_ref[1] = gr
    hp = (jnp.dot(gl, W_ref[:H, :], preferred_element_type=jnp.float32)
          + jnp.dot(gr, W_ref[H:, :], preferred_element_type=jnp.float32))
    _proj_tail(hp, h_ref, s_ref, d_ref, as_ref, ad_ref)


def _ep_body(num_ref, den_ref, b_ref, gp_ref, out_ref):
    inv = 1.0 / (den_ref[...] + EPS)
    out_ref[:, :H] = jnp.maximum(num_ref[0] * inv + b_ref[0:1, :H], 0.0) + gp_ref[0]
    out_ref[:, H:] = jnp.maximum(num_ref[1] * inv + b_ref[0:1, H:], 0.0) + gp_ref[1]


_GRID = N // BLK

_full = lambda *shape: pl.BlockSpec(shape, lambda i: tuple(0 for _ in shape))
_rows1 = pl.BlockSpec((BLK, 1), lambda i: (i, 0))
_rows2 = pl.BlockSpec((2, BLK, H), lambda i: (0, i, 0))
_rowsD = pl.BlockSpec((BLK, D), lambda i: (i, 0))

_state_out = [
    jax.ShapeDtypeStruct((2, N, H), jnp.float32),
    jax.ShapeDtypeStruct((2, N, H), jnp.float32),
    jax.ShapeDtypeStruct((N, 1), jnp.float32),
    jax.ShapeDtypeStruct((N, 1), jnp.float32),
]

_l1_call = pl.pallas_call(
    _l1_body,
    grid=(_GRID,),
    in_specs=[_rows1, _full(32, D), _full(D, D), _full(D, 1), _full(D, 1)],
    out_specs=[_rows2, _rows2, _rows1, _rows1],
    out_shape=_state_out,
)

_mid_call = pl.pallas_call(
    _mid_body,
    grid=(_GRID,),
    in_specs=[_rows2, _rows1, _full(1, D), _rows2, _full(D, D), _full(D, 1), _full(D, 1)],
    out_specs=[_rows2, _rows2, _rows1, _rows1],
    out_shape=_state_out,
)

_ep_call = pl.pallas_call(
    _ep_body,
    grid=(_GRID,),
    in_specs=[_rows2, _rows1, _full(1, D), _rows2],
    out_specs=_rowsD,
    out_shape=jax.ShapeDtypeStruct((N, D), jnp.float32),
)


_mesh = plsc.VectorSubcoreMesh(core_axis_name="c", subcore_axis_name="s")


@functools.partial(
    pl.kernel,
    out_type=[
        jax.ShapeDtypeStruct((2, N, H), jnp.float32),
        jax.ShapeDtypeStruct((N,), jnp.float32),
    ],
    mesh=_mesh,
    compiler_params=pltpu.CompilerParams(needs_layout_passes=False),
    scratch_types=[
        pltpu.VMEM((NBATCH, B), jnp.int32),
        pltpu.VMEM((NBATCH, B), jnp.int32),
        pltpu.VMEM((B,), jnp.float32),
        pltpu.VMEM((B,), jnp.float32),
        pltpu.VMEM((B,), jnp.float32),
        pltpu.VMEM((B,), jnp.float32),
        pltpu.VMEM((B, H), jnp.float32),
        pltpu.VMEM((1024,), jnp.float32),
        pltpu.VMEM_SHARED((N, H), jnp.float32),
        pltpu.VMEM_SHARED((N,), jnp.float32),
    ],
)
def _sc_edge(h2, s_in, d_in, src1, dst1, src3, dst3, ew1,
             num_out, den_out,
             src2d, dst2d, ewb, sg, dg, eb, buf, zflat, acc_s, den_s):
    c = lax.axis_index("c")
    sid = lax.axis_index("s")
    base_e = sid * T

    pltpu.sync_copy(src3.at[sid], src2d)
    pltpu.sync_copy(dst3.at[sid], dst2d)

    zero16 = jnp.zeros((16,), jnp.float32)

    @pl.loop(0, B)
    def _zbuf(t):
        for q in range(8):
            buf[t, pl.ds(q * 16, 16)] = zero16

    @pl.loop(0, 64)
    def _zflat(t):
        zflat[pl.ds(t * 16, 16)] = zero16

    row0 = sid * 624 + jnp.minimum(sid, 2) * 8

    def _rows_chunks(count):
        full, rem = divmod(count, 80)
        return [(k * 80, 80) for k in range(full)] + ([(full * 80, rem)] if rem else [])

    def _per_tile_rows(fn):
        for cnt in (632, 624):
            @pl.when(sid < 2 if cnt == 632 else sid >= 2)
            def _go(cnt=cnt):
                for off, n in _rows_chunks(cnt):
                    fn(row0 + off, n)

    _per_tile_rows(lambda r, n: pltpu.sync_copy(buf.at[pl.ds(0, n)],
                                                acc_s.at[pl.ds(r, n)]))

    @pl.when(jnp.logical_and(c == 0, sid < 10))
    def _zden():
        pltpu.sync_copy(zflat.at[pl.ds(0, 1000)], den_s.at[pl.ds(sid * 1000, 1000)])

    plsc.subcore_barrier()

    h_half = h2.at[c]

    @pl.loop(0, NBATCH)
    def _batch(j):
        pltpu.sync_copy(h_half.at[src2d.at[j]], buf)
        pltpu.sync_copy(s_in.at[src2d.at[j]], sg)
        pltpu.sync_copy(d_in.at[dst2d.at[j]], dg)
        pltpu.sync_copy(ew1.at[pl.ds(base_e + j * B, B)], ewb)

        for k in range(5):
            off = k * 16
            lg = sg[pl.ds(off, 16)] + dg[pl.ds(off, 16)]
            lg = jnp.where(lg >= 0, lg, NEG * lg)
            ev = jnp.exp(lg)
            eb[pl.ds(off, 16)] = ev
            uv = ev * ewb[pl.ds(off, 16)]
            for l in range(16):
                u = uv[l]
                t = off + l
                for q in range(8):
                    buf[t, pl.ds(q * 16, 16)] = buf[t, pl.ds(q * 16, 16)] * u

        pltpu.sync_copy(buf, acc_s.at[dst2d.at[j]], add=True)

        @pl.when(c == 0)
        def _den():
            pltpu.sync_copy(eb, den_s.at[dst2d.at[j]], add=True)

    plsc.subcore_barrier()

    acc_out = num_out.at[c]

    def _ep(r, n):
        pltpu.sync_copy(acc_s.at[pl.ds(r, n)], buf.at[pl.ds(0, n)])
        pltpu.sync_copy(buf.at[pl.ds(0, n)], acc_out.at[pl.ds(r, n)])

    _per_tile_rows(_ep)

    @pl.when(jnp.logical_and(c == 0, sid < 10))
    def _wden():
        pltpu.sync_copy(den_s.at[pl.ds(sid * 1000, 1000)], zflat.at[pl.ds(0, 1000)])
        pltpu.sync_copy(zflat.at[pl.ds(0, 1000)], den_out.at[pl.ds(sid * 1000, 1000)])


def kernel(x, edge_index, edge_attr, emb, W1, a1s, a1d, b1, W2, a2s, a2d, b2,
           W3, a3s, a3d, b3):
    src1 = edge_index[0]
    dst1 = edge_index[1]
    src3 = src1.reshape(NS, NBATCH, B)
    dst3 = dst1.reshape(NS, NBATCH, B)
    ew1 = edge_attr.astype(jnp.float32)

    embp = jnp.pad(emb, ((0, 32 - emb.shape[0]), (0, 0)))
    col = lambda v: v.reshape(D, 1)
    rowv = lambda v: v.reshape(1, D)

    g, h, s, d = _l1_call(x, embp, W1, col(a1s), col(a1d))
    num, den = _sc_edge(h, s.reshape(N), d.reshape(N), src1, dst1, src3, dst3, ew1)

    g, h, s, d = _mid_call(num, den.reshape(N, 1), rowv(b1), g, W2, col(a2s), col(a2d))
    num, den = _sc_edge(h, s.reshape(N), d.reshape(N), src1, dst1, src3, dst3, ew1)

    g, h, s, d = _mid_call(num, den.reshape(N, 1), rowv(b2), g, W3, col(a3s), col(a3d))
    num, den = _sc_edge(h, s.reshape(N), d.reshape(N), src1, dst1, src3, dst3, ew1)

    return _ep_call(num, den.reshape(N, 1), rowv(b3), g)
